# baseline (speedup 1.0000x reference)
import functools

import jax
import jax.numpy as jnp
from jax import lax
from jax.experimental import pallas as pl
from jax.experimental.pallas import tpu as pltpu
from jax.experimental.pallas import tpu_sc as plsc

_B = 16384
_D = 32
_NC = 2
_NS = 16
_NW = _NC * _NS
_BPW = _B // _NW
_CHUNK = 128
_NCHUNK = _BPW // _CHUNK

_mesh = plsc.VectorSubcoreMesh(core_axis_name="c", subcore_axis_name="s")


@functools.partial(
    pl.kernel,
    mesh=_mesh,
    compiler_params=pltpu.CompilerParams(use_tc_tiling_on_sc=False),
    out_type=jax.ShapeDtypeStruct((_D, _B), jnp.float32),
    scratch_types=[
        pltpu.VMEM((_NCHUNK, _CHUNK), jnp.int32),
        pltpu.VMEM((_NCHUNK, _CHUNK), jnp.int32),
        pltpu.VMEM((2, _BPW), jnp.float32),
        pltpu.VMEM((2, _BPW), jnp.float32),
        pltpu.VMEM((_D, _BPW), jnp.float32),
        pltpu.SemaphoreType.DMA,
        pltpu.SemaphoreType.DMA,
    ],
)
def _funk_fwd(item_hbm, user_hbm, items_t, users_t, out_t,
              iidx, uidx, ibuf, ubuf, obuf, sem_i, sem_u):
    wid = lax.axis_index("s") * _NC + lax.axis_index("c")
    base = wid * _BPW

    for c in range(_NCHUNK):
        pltpu.sync_copy(item_hbm.at[pl.ds(base + c * _CHUNK, _CHUNK)],
                        iidx.at[c])
        pltpu.sync_copy(user_hbm.at[pl.ds(base + c * _CHUNK, _CHUNK)],
                        uidx.at[c])

    def fire(d, slot):
        for c in range(_NCHUNK):
            sl = pl.ds(c * _CHUNK, _CHUNK)
            pltpu.async_copy(items_t.at[d].at[iidx.at[c]],
                             ibuf.at[slot, sl], sem_i)
            pltpu.async_copy(users_t.at[d].at[uidx.at[c]],
                             ubuf.at[slot, sl], sem_u)

    def drain(d, slot):
        for c in range(_NCHUNK):
            sl = pl.ds(c * _CHUNK, _CHUNK)
            pltpu.make_async_copy(items_t.at[d].at[iidx.at[c]],
                                  ibuf.at[slot, sl], sem_i).wait()
            pltpu.make_async_copy(users_t.at[d].at[uidx.at[c]],
                                  ubuf.at[slot, sl], sem_u).wait()

    fire(0, 0)
    fire(1, 1)

    def plane_body(d, carry):
        slot = lax.rem(d, 2)
        drain(d, slot)

        @pl.when(d < _D - 2)
        def _():
            fire(d + 2, slot)

        def mul_body(k, c2):
            sl = pl.ds(k * 16, 16)
            obuf[d, sl] = ibuf[slot, sl] * ubuf[slot, sl]
            return c2

        lax.fori_loop(0, _BPW // 16, mul_body, 0, unroll=8)
        return carry

    lax.fori_loop(0, _D, plane_body, 0)

    pltpu.sync_copy(obuf, out_t.at[:, pl.ds(base, _BPW)])


def kernel(item, user, users, items):
    out_t = _funk_fwd(item, user, items.T, users.T)
    return out_t.T


# speedup vs baseline: 5.6297x; 5.6297x over previous
import functools

import jax
import jax.numpy as jnp
from jax import lax
from jax.experimental import pallas as pl
from jax.experimental.pallas import tpu as pltpu
from jax.experimental.pallas import tpu_sc as plsc

_B = 16384
_D = 32
_NC = 2
_NS = 16
_NW = _NC * _NS
_BPW = _B // _NW
_CHUNK = 128
_NCHUNK = _BPW // _CHUNK

_mesh = plsc.VectorSubcoreMesh(core_axis_name="c", subcore_axis_name="s")


@functools.partial(
    pl.kernel,
    mesh=_mesh,
    compiler_params=pltpu.CompilerParams(use_tc_tiling_on_sc=False),
    out_type=jax.ShapeDtypeStruct((_B, _D), jnp.float32),
    scratch_types=[
        pltpu.VMEM((_BPW,), jnp.int32),
        pltpu.VMEM((_BPW,), jnp.int32),
        pltpu.VMEM((_BPW, _D), jnp.float32),
        pltpu.VMEM((_BPW, _D), jnp.float32),
        pltpu.SemaphoreType.DMA,
        pltpu.SemaphoreType.DMA,
    ],
)
def _funk_fwd(item_hbm, user_hbm, items_hbm, users_hbm, out_hbm,
              iidx, uidx, irows, urows, sem_i, sem_u):
    wid = lax.axis_index("s") * _NC + lax.axis_index("c")
    base = wid * _BPW

    pltpu.sync_copy(item_hbm.at[pl.ds(base, _BPW)], iidx)
    pltpu.sync_copy(user_hbm.at[pl.ds(base, _BPW)], uidx)

    copies = []
    for c in range(_NCHUNK):
        sl = pl.ds(c * _CHUNK, _CHUNK)
        copies.append(
            pltpu.async_copy(items_hbm.at[iidx.at[sl]], irows.at[sl], sem_i))
        copies.append(
            pltpu.async_copy(users_hbm.at[uidx.at[sl]], urows.at[sl], sem_u))
    for cp in copies:
        cp.wait()

    def body(i, carry):
        for h in range(_D // 16):
            sl = pl.ds(h * 16, 16)
            irows[i, sl] = irows[i, sl] * urows[i, sl]
        return carry

    lax.fori_loop(0, _BPW, body, 0, unroll=4)

    pltpu.sync_copy(irows, out_hbm.at[pl.ds(base, _BPW)])


def kernel(item, user, users, items):
    return _funk_fwd(item, user, items, users)
